# Initial kernel scaffold; baseline (speedup 1.0000x reference)
import jax
import jax.numpy as jnp
from jax.experimental import pallas as pl


def kernel(features_list, lut):
    raise NotImplementedError("write your pallas kernel here")


# speedup vs baseline: 1.3114x; 1.3114x over previous
import functools

import jax
import jax.numpy as jnp
from jax import lax
from jax.experimental import pallas as pl
from jax.experimental.pallas import tpu as pltpu
from jax.experimental.pallas import tpu_sc as plsc

_CHUNK = 128


def _sc_gather(table, idx):
    info = plsc.get_sparse_core_info()
    nw = info.num_cores * info.num_subcores
    nv = idx.shape[0]
    d = table.shape[1]
    per_w = nv // nw
    assert per_w * nw == nv and per_w % 8 == 0
    n_full = per_w // _CHUNK
    tail = per_w - n_full * _CHUNK

    mesh = plsc.VectorSubcoreMesh(core_axis_name="c", subcore_axis_name="s")

    @functools.partial(
        pl.kernel,
        mesh=mesh,
        compiler_params=pltpu.CompilerParams(use_tc_tiling_on_sc=False),
        out_type=jax.ShapeDtypeStruct((nv, d), jnp.float32),
        scratch_types=[
            pltpu.VMEM((_CHUNK,), jnp.int32),
            pltpu.VMEM((_CHUNK, d), jnp.float32),
            pltpu.SemaphoreType.DMA,
        ],
    )
    def gather_kernel(table_hbm, idx_hbm, out_hbm, idx_v, rows_v, sem):
        wid = lax.axis_index("s") * info.num_cores + lax.axis_index("c")
        base = wid * per_w

        def do_chunk(off):
            pltpu.sync_copy(idx_hbm.at[pl.ds(off, _CHUNK)], idx_v)
            pltpu.async_copy(table_hbm.at[idx_v], rows_v, sem).wait()
            pltpu.sync_copy(rows_v, out_hbm.at[pl.ds(off, _CHUNK)])

        def body(j, carry):
            do_chunk(base + j * _CHUNK)
            return carry

        lax.fori_loop(0, n_full, body, 0)
        if tail:
            do_chunk(base + per_w - _CHUNK)

    return gather_kernel(table, idx)


def _tc_transpose(g):
    nv, d = g.shape
    vb = 512
    assert nv % vb == 0

    def body(in_ref, out_ref):
        out_ref[...] = in_ref[...].T

    return pl.pallas_call(
        body,
        grid=(nv // vb,),
        in_specs=[pl.BlockSpec((vb, d), lambda i: (i, 0))],
        out_specs=pl.BlockSpec((d, vb), lambda i: (0, i)),
        out_shape=jax.ShapeDtypeStruct((d, nv), jnp.float32),
    )(g)


def kernel(features_list, lut):
    feat = features_list[0]
    b, n, c, h, w = feat.shape
    nv = lut.shape[0]
    nz = 16
    nx = ny = 200

    table = jnp.transpose(feat[0], (0, 2, 3, 1)).reshape(n * h * w, c)
    table = jnp.concatenate([table, jnp.zeros((8, c), table.dtype)], axis=0)

    lut32 = lut.astype(jnp.int32)
    valid = lut32[:, 0] >= 0
    flat = lut32[:, 0] * (h * w) + lut32[:, 2] * w + lut32[:, 1]
    idx = jnp.where(valid, flat, n * h * w).astype(jnp.int32)

    gathered = _sc_gather(table, idx)
    out_t = _tc_transpose(gathered)
    return (out_t.reshape(1, c, nx, ny, nz),)


# speedup vs baseline: 1.3584x; 1.0358x over previous
import functools

import jax
import jax.numpy as jnp
from jax import lax
from jax.experimental import pallas as pl
from jax.experimental.pallas import tpu as pltpu
from jax.experimental.pallas import tpu_sc as plsc

_CHUNK = 128


def _sc_gather(table, idx):
    info = plsc.get_sparse_core_info()
    nw = info.num_cores * info.num_subcores
    nv = idx.shape[0]
    d = table.shape[1]
    per_w = nv // nw
    assert per_w * nw == nv and per_w % 8 == 0
    ch = _CHUNK
    nb = 4
    n_chunks = per_w // ch
    tail = per_w - n_chunks * ch
    assert n_chunks >= 2 * nb and tail % 8 == 0

    mesh = plsc.VectorSubcoreMesh(core_axis_name="c", subcore_axis_name="s")

    @functools.partial(
        pl.kernel,
        mesh=mesh,
        compiler_params=pltpu.CompilerParams(use_tc_tiling_on_sc=False),
        out_type=jax.ShapeDtypeStruct((nv, d), jnp.float32),
        scratch_types=[
            pltpu.VMEM((nb, ch), jnp.int32),
            pltpu.VMEM((nb, ch, d), jnp.float32),
            pltpu.SemaphoreType.DMA((nb,)),
            pltpu.SemaphoreType.DMA((nb,)),
            pltpu.SemaphoreType.DMA((nb,)),
        ],
    )
    def gather_kernel(table_hbm, idx_hbm, out_hbm, idx_v, rows_v, sem_i, sem_g, sem_o):
        wid = lax.axis_index("s") * info.num_cores + lax.axis_index("c")
        base = wid * per_w

        def off(j):
            return base + j * ch

        def start_idx(j, b):
            pltpu.async_copy(idx_hbm.at[pl.ds(off(j), ch)], idx_v.at[b], sem_i.at[b])

        def wait_idx(b):
            pltpu.make_async_copy(idx_hbm.at[pl.ds(base, ch)], idx_v.at[b], sem_i.at[b]).wait()

        def start_gather(j, b):
            del j
            pltpu.async_copy(table_hbm.at[idx_v.at[b]], rows_v.at[b], sem_g.at[b])

        def wait_gather(b):
            pltpu.make_async_copy(table_hbm.at[idx_v.at[b]], rows_v.at[b], sem_g.at[b]).wait()

        def start_out(j, b):
            pltpu.async_copy(rows_v.at[b], out_hbm.at[pl.ds(off(j), ch)], sem_o.at[b])

        def wait_out(b):
            pltpu.make_async_copy(rows_v.at[b], out_hbm.at[pl.ds(base, ch)], sem_o.at[b]).wait()

        def body(j, b, first_round, gather_next, prefetch):
            nxt = (b + 1) % nb
            if not first_round:
                wait_out(nxt)
            if gather_next:
                wait_idx(nxt)
                start_gather(j + 1, nxt)
            wait_gather(b)
            start_out(j, b)
            if prefetch:
                start_idx(j + nb, b)

        for b in range(nb):
            start_idx(b, b)
        wait_idx(0)
        start_gather(0, 0)
        for b in range(nb):
            body(b, b, first_round=(b + 1 < nb), gather_next=True, prefetch=True)

        n_groups = n_chunks // nb
        def group(g, carry):
            j0 = g * nb
            for b in range(nb):
                body(j0 + b, b, first_round=False, gather_next=True, prefetch=True)
            return carry
        lax.fori_loop(1, n_groups - 1, group, 0)

        for j in range((n_groups - 1) * nb, n_chunks):
            body(j, j % nb, first_round=False,
                 gather_next=(j + 1 < n_chunks),
                 prefetch=(j + nb < n_chunks))
        for j in range(n_chunks - nb + 1, n_chunks):
            wait_out(j % nb)

        if tail:
            toff = base + n_chunks * ch
            pltpu.sync_copy(idx_hbm.at[pl.ds(toff, tail)], idx_v.at[0, pl.ds(0, tail)])
            pltpu.async_copy(
                table_hbm.at[idx_v.at[0, pl.ds(0, tail)]],
                rows_v.at[0, pl.ds(0, tail)], sem_g.at[0]).wait()
            pltpu.sync_copy(rows_v.at[0, pl.ds(0, tail)], out_hbm.at[pl.ds(toff, tail)])

    return gather_kernel(table, idx)


def _tc_transpose(g):
    nv, d = g.shape
    vb = 512
    assert nv % vb == 0

    def body(in_ref, out_ref):
        out_ref[...] = in_ref[...].T

    return pl.pallas_call(
        body,
        grid=(nv // vb,),
        in_specs=[pl.BlockSpec((vb, d), lambda i: (i, 0))],
        out_specs=pl.BlockSpec((d, vb), lambda i: (0, i)),
        out_shape=jax.ShapeDtypeStruct((d, nv), jnp.float32),
    )(g)


def kernel(features_list, lut):
    feat = features_list[0]
    b, n, c, h, w = feat.shape
    nv = lut.shape[0]
    nz = 16
    nx = ny = 200

    table = jnp.transpose(feat[0], (0, 2, 3, 1)).reshape(n * h * w, c)
    table = jnp.concatenate([table, jnp.zeros((8, c), table.dtype)], axis=0)

    lut32 = lut.astype(jnp.int32)
    valid = lut32[:, 0] >= 0
    flat = lut32[:, 0] * (h * w) + lut32[:, 2] * w + lut32[:, 1]
    idx = jnp.where(valid, flat, n * h * w).astype(jnp.int32)

    gathered = _sc_gather(table, idx)
    out_t = _tc_transpose(gathered)
    return (out_t.reshape(1, c, nx, ny, nz),)
